# baseline (speedup 1.0000x reference)
import jax
import jax.numpy as jnp
from jax import lax
from jax.experimental import pallas as pl
from jax.experimental.pallas import tpu as pltpu
from jax.experimental.pallas import tpu_sc as plsc

N = 100000
D = 128
MAXDEG = 512
ROWS = MAXDEG + 1
PKW = D // 2
NW = 32
NB = 160
NBLK = N // NB
HI = NBLK % NW
BLK_LO = NBLK // NW
BLK_HImax = BLK_LO + 1
PMAX = (BLK_HImax + 1) // 2
IDXMAX = BLK_HImax * NB


def _pack_table(w):
  wb = w.astype(jnp.bfloat16).reshape(ROWS, 4, 2, 16)
  u = lax.bitcast_convert_type(wb, jnp.uint16).astype(jnp.uint32)
  packed = u[:, :, 0, :] | (u[:, :, 1, :] << 16)
  return lax.bitcast_convert_type(packed, jnp.int32).reshape(ROWS * PKW)


def _body(nfeats, ind, outd, wa, wb, out,
          tab_a, tab_b, idx_a, idx_b, feats0, feats1,
          sem_tab, sem_in0, sem_in1, sem_out0, sem_out1):
  c = lax.axis_index("c")
  s = lax.axis_index("s")
  wid = s * 2 + c
  nblk = jnp.where(wid < HI, BLK_HImax, BLK_LO)
  sblk = wid * BLK_LO + jnp.minimum(wid, HI)
  nbase = sblk * NB

  cp1 = pltpu.async_copy(wa, tab_a, sem_tab)
  cp2 = pltpu.async_copy(wb, tab_b, sem_tab)

  @pl.when(nblk == BLK_HImax)
  def _():
    pltpu.async_copy(ind.at[pl.ds(nbase, IDXMAX)], idx_a, sem_tab)
    pltpu.async_copy(outd.at[pl.ds(nbase, IDXMAX)], idx_b, sem_tab)

  @pl.when(nblk == BLK_LO)
  def _():
    pltpu.async_copy(ind.at[pl.ds(nbase, BLK_LO * NB)],
                     idx_a.at[pl.ds(0, BLK_LO * NB)], sem_tab)
    pltpu.async_copy(outd.at[pl.ds(nbase, BLK_LO * NB)],
                     idx_b.at[pl.ds(0, BLK_LO * NB)], sem_tab)

  cp1.wait()
  cp2.wait()

  @pl.when(nblk == BLK_HImax)
  def _():
    pltpu.make_async_copy(ind.at[pl.ds(0, IDXMAX)], idx_a, sem_tab).wait()
    pltpu.make_async_copy(outd.at[pl.ds(0, IDXMAX)], idx_b, sem_tab).wait()

  @pl.when(nblk == BLK_LO)
  def _():
    pltpu.make_async_copy(ind.at[pl.ds(0, BLK_LO * NB)],
                          idx_a.at[pl.ds(0, BLK_LO * NB)], sem_tab).wait()
    pltpu.make_async_copy(outd.at[pl.ds(0, BLK_LO * NB)],
                          idx_b.at[pl.ds(0, BLK_LO * NB)], sem_tab).wait()

  cols = [lax.iota(jnp.int32, 16) + (g * 16) for g in range(4)]

  def fire_in(j, ft, sem):
    pltpu.async_copy(nfeats.at[pl.ds((sblk + j) * NB, NB)], ft, sem)

  def wait_in(ft, sem):
    pltpu.make_async_copy(nfeats.at[pl.ds(0, NB)], ft, sem).wait()

  def fire_out(j, ft, sem):
    pltpu.async_copy(ft, out.at[pl.ds((sblk + j) * NB, NB)], sem)

  def wait_out(ft, sem):
    pltpu.make_async_copy(ft, out.at[pl.ds(0, NB)], sem).wait()

  def compute(j, ft):
    joff = j * NB

    @plsc.parallel_loop(0, NB // 16, 1, unroll=2)
    def _chunk(m):
      iva = idx_a[pl.ds(joff + m * 16, 16)] * PKW
      ivb = idx_b[pl.ds(joff + m * 16, 16)] * PKW
      for lane in range(16):
        n = m * 16 + lane
        ra = iva[lane]
        rb = ivb[lane]
        for g in range(4):
          pa = plsc.load_gather(tab_a, [cols[g] + ra])
          pb = plsc.load_gather(tab_b, [cols[g] + rb])
          a0, a1 = plsc.unpack(plsc.bitcast(pa, jnp.bfloat16),
                               format=plsc.PackFormat.INTERLEAVED)
          b0, b1 = plsc.unpack(plsc.bitcast(pb, jnp.bfloat16),
                               format=plsc.PackFormat.INTERLEAVED)
          plsc.addupdate(ft.at[n, pl.ds(g * 32, 16)], a0 + b0)
          plsc.addupdate(ft.at[n, pl.ds(g * 32 + 16, 16)], a1 + b1)

  fire_in(0, feats0, sem_in0)

  def pair(p, carry):
    j0 = p * 2
    j1 = j0 + 1

    @pl.when(p > 0)
    def _():
      wait_out(feats1, sem_out1)

    @pl.when(j1 < nblk)
    def _():
      fire_in(j1, feats1, sem_in1)

    wait_in(feats0, sem_in0)
    compute(j0, feats0)
    fire_out(j0, feats0, sem_out0)

    @pl.when(j1 < nblk)
    def _():
      wait_in(feats1, sem_in1)
      compute(j1, feats1)
      fire_out(j1, feats1, sem_out1)

    wait_out(feats0, sem_out0)

    @pl.when(j0 + 2 < nblk)
    def _():
      fire_in(j0 + 2, feats0, sem_in0)

    return carry

  lax.fori_loop(0, PMAX, pair, 0)

  @pl.when(nblk == BLK_HImax)
  def _():
    wait_out(feats1, sem_out1)


@jax.jit
def kernel(nfeats, in_degrees, out_degrees, W_in, W_out):
  ind = jnp.clip(in_degrees, 0, MAXDEG).astype(jnp.int32)
  outd = jnp.clip(out_degrees, 0, MAXDEG).astype(jnp.int32)
  wa = _pack_table(W_in)
  wb = _pack_table(W_out)
  mesh = plsc.VectorSubcoreMesh(core_axis_name="c", subcore_axis_name="s")
  f = pl.kernel(
      _body,
      out_type=jax.ShapeDtypeStruct((N, D), jnp.float32),
      mesh=mesh,
      compiler_params=pltpu.CompilerParams(needs_layout_passes=False),
      scratch_types=[
          pltpu.VMEM((ROWS * PKW,), jnp.int32),
          pltpu.VMEM((ROWS * PKW,), jnp.int32),
          pltpu.VMEM((IDXMAX,), jnp.int32),
          pltpu.VMEM((IDXMAX,), jnp.int32),
          pltpu.VMEM((NB, D), jnp.float32),
          pltpu.VMEM((NB, D), jnp.float32),
          pltpu.SemaphoreType.DMA,
          pltpu.SemaphoreType.DMA,
          pltpu.SemaphoreType.DMA,
          pltpu.SemaphoreType.DMA,
          pltpu.SemaphoreType.DMA,
      ],
  )
  return f(nfeats, ind, outd, wa, wb)


# speedup vs baseline: 1.1443x; 1.1443x over previous
import jax
import jax.numpy as jnp
from jax import lax
from jax.experimental import pallas as pl
from jax.experimental.pallas import tpu as pltpu
from jax.experimental.pallas import tpu_sc as plsc

N = 100000
D = 128
MAXDEG = 512
ROWS = MAXDEG + 1
PKW = D // 2
NW = 32
NB = 160
NBLK = N // NB
HI = NBLK % NW
BLK_LO = NBLK // NW
BLK_HImax = BLK_LO + 1
NSLOT3 = (BLK_HImax + 2) // 3


def _pack_table(w):
  wb = w.astype(jnp.bfloat16).reshape(ROWS, 4, 2, 16)
  u = lax.bitcast_convert_type(wb, jnp.uint16).astype(jnp.uint32)
  packed = u[:, :, 0, :] | (u[:, :, 1, :] << 16)
  return lax.bitcast_convert_type(packed, jnp.int32).reshape(ROWS * PKW)


def _body(nfeats, ind, outd, wa, wb, out,
          tab_a, tab_b,
          ia0, ib0, ft0, ia1, ib1, ft1, ia2, ib2, ft2,
          sem_tab, sem_in0, sem_in1, sem_in2, sem_out0, sem_out1, sem_out2):
  c = lax.axis_index("c")
  s = lax.axis_index("s")
  wid = s * 2 + c
  nblk = jnp.where(wid < HI, BLK_HImax, BLK_LO)
  sblk = wid * BLK_LO + jnp.minimum(wid, HI)

  bufs = [(ia0, ib0, ft0, sem_in0, sem_out0),
          (ia1, ib1, ft1, sem_in1, sem_out1),
          (ia2, ib2, ft2, sem_in2, sem_out2)]

  cp1 = pltpu.async_copy(wa, tab_a, sem_tab)
  cp2 = pltpu.async_copy(wb, tab_b, sem_tab)
  cp1.wait()
  cp2.wait()

  cols = [lax.iota(jnp.int32, 16) + (g * 16) for g in range(4)]

  def fire_in(j, b):
    ia, ib, ft, sem, _ = bufs[b]
    base = (sblk + j) * NB
    pltpu.async_copy(ind.at[pl.ds(base, NB)], ia, sem)
    pltpu.async_copy(outd.at[pl.ds(base, NB)], ib, sem)
    pltpu.async_copy(nfeats.at[pl.ds(base, NB)], ft, sem)

  def wait_in(b):
    ia, ib, ft, sem, _ = bufs[b]
    pltpu.make_async_copy(ind.at[pl.ds(0, NB)], ia, sem).wait()
    pltpu.make_async_copy(outd.at[pl.ds(0, NB)], ib, sem).wait()
    pltpu.make_async_copy(nfeats.at[pl.ds(0, NB)], ft, sem).wait()

  def fire_out(j, b):
    _, _, ft, _, sem = bufs[b]
    pltpu.async_copy(ft, out.at[pl.ds((sblk + j) * NB, NB)], sem)

  def wait_out(b):
    _, _, ft, _, sem = bufs[b]
    pltpu.make_async_copy(ft, out.at[pl.ds(0, NB)], sem).wait()

  def compute(b):
    ia, ib, ft, _, _ = bufs[b]

    @plsc.parallel_loop(0, NB // 16, 1, unroll=2)
    def _chunk(m):
      iva = ia[pl.ds(m * 16, 16)] * PKW
      ivb = ib[pl.ds(m * 16, 16)] * PKW
      for lane in range(16):
        n = m * 16 + lane
        ra = iva[lane]
        rb = ivb[lane]
        for g in range(4):
          pa = plsc.load_gather(tab_a, [cols[g] + ra])
          pb = plsc.load_gather(tab_b, [cols[g] + rb])
          a0, a1 = plsc.unpack(plsc.bitcast(pa, jnp.bfloat16),
                               format=plsc.PackFormat.INTERLEAVED)
          b0, b1 = plsc.unpack(plsc.bitcast(pb, jnp.bfloat16),
                               format=plsc.PackFormat.INTERLEAVED)
          plsc.addupdate(ft.at[n, pl.ds(g * 32, 16)], a0 + b0)
          plsc.addupdate(ft.at[n, pl.ds(g * 32 + 16, 16)], a1 + b1)

  fire_in(0, 0)
  fire_in(1, 1)

  def turn(k, carry):
    for b in range(3):
      j = k * 3 + b

      @pl.when(j < nblk)
      def _():
        wait_in(b)
        compute(b)

        @pl.when(j >= 1)
        def _():
          wait_out((b + 2) % 3)

        @pl.when(j + 2 < nblk)
        def _():
          fire_in(j + 2, (b + 2) % 3)

        fire_out(j, b)

    return carry

  lax.fori_loop(0, NSLOT3, turn, 0)

  @pl.when(nblk == BLK_LO)
  def _():
    wait_out((BLK_LO - 1) % 3)

  @pl.when(nblk == BLK_HImax)
  def _():
    wait_out((BLK_HImax - 1) % 3)


@jax.jit
def kernel(nfeats, in_degrees, out_degrees, W_in, W_out):
  ind = jnp.clip(in_degrees, 0, MAXDEG).astype(jnp.int32)
  outd = jnp.clip(out_degrees, 0, MAXDEG).astype(jnp.int32)
  wa = _pack_table(W_in)
  wb = _pack_table(W_out)
  mesh = plsc.VectorSubcoreMesh(core_axis_name="c", subcore_axis_name="s")
  f = pl.kernel(
      _body,
      out_type=jax.ShapeDtypeStruct((N, D), jnp.float32),
      mesh=mesh,
      compiler_params=pltpu.CompilerParams(needs_layout_passes=False),
      scratch_types=[
          pltpu.VMEM((ROWS * PKW,), jnp.int32),
          pltpu.VMEM((ROWS * PKW,), jnp.int32),
          pltpu.VMEM((NB,), jnp.int32),
          pltpu.VMEM((NB,), jnp.int32),
          pltpu.VMEM((NB, D), jnp.float32),
          pltpu.VMEM((NB,), jnp.int32),
          pltpu.VMEM((NB,), jnp.int32),
          pltpu.VMEM((NB, D), jnp.float32),
          pltpu.VMEM((NB,), jnp.int32),
          pltpu.VMEM((NB,), jnp.int32),
          pltpu.VMEM((NB, D), jnp.float32),
          pltpu.SemaphoreType.DMA,
          pltpu.SemaphoreType.DMA,
          pltpu.SemaphoreType.DMA,
          pltpu.SemaphoreType.DMA,
          pltpu.SemaphoreType.DMA,
          pltpu.SemaphoreType.DMA,
          pltpu.SemaphoreType.DMA,
      ],
  )
  return f(nfeats, ind, outd, wa, wb)
